# Initial kernel scaffold; baseline (speedup 1.0000x reference)
import jax
import jax.numpy as jnp
from jax.experimental import pallas as pl


def kernel(prompts, prompt_types, Wp, bp, Wb, bb, W1, b1, g1, be1, W2, b2, type_emb):
    raise NotImplementedError("write your pallas kernel here")


# speedup vs baseline: 1.2360x; 1.2360x over previous
import functools

import jax
import jax.numpy as jnp
from jax.experimental import pallas as pl
from jax.experimental.pallas import tpu as pltpu

B, N, DMAX = 64, 2048, 64
D = 128
BLK = 2048


def _body(x_ref, t_ref, wpb_ref, w1_ref, w2_ref, cp_ref, cb_ref, c1_ref,
          c2_ref, g1_ref, be1_ref, o_ref):
    x = x_ref[:, :]
    t = t_ref[:, :]

    h = jnp.dot(x, w1_ref[:, :], preferred_element_type=jnp.float32)
    h = h + c1_ref[0, :]
    mu = jnp.mean(h, axis=-1, keepdims=True)
    var = jnp.mean((h - mu) ** 2, axis=-1, keepdims=True)
    h = (h - mu) * jax.lax.rsqrt(var + 1e-5) * g1_ref[0, :] + be1_ref[0, :]
    h = jnp.maximum(h, 0.0)
    poly = jnp.dot(h, w2_ref[:, :], preferred_element_type=jnp.float32)
    poly = poly + c2_ref[0, :]

    x8 = x[:, :8]
    wpb = wpb_ref[:, :]
    pt = jnp.dot(x8, wpb[:8, :], preferred_element_type=jnp.float32) + cp_ref[0, :]
    bx = jnp.dot(x8, wpb[8:, :], preferred_element_type=jnp.float32) + cb_ref[0, :]

    o_ref[:, :] = jnp.where(t == 0, pt, jnp.where(t == 1, bx, poly))


def kernel(prompts, prompt_types, Wp, bp, Wb, bb, W1, b1, g1, be1, W2, b2,
           type_emb):
    BN = B * N
    x = prompts.reshape(BN, DMAX)
    t = prompt_types.reshape(BN, 1)

    wp8 = jnp.zeros((8, D), jnp.float32).at[:2, :].set(Wp)
    wb8 = jnp.zeros((8, D), jnp.float32).at[:4, :].set(Wb)
    wpb = jnp.concatenate([wp8, wb8], axis=0)

    cp = (bp + type_emb[0]).reshape(1, D)
    cb = (bb + type_emb[1]).reshape(1, D)
    c1 = b1.reshape(1, D)
    c2 = (b2 + type_emb[2]).reshape(1, D)
    g1r = g1.reshape(1, D)
    be1r = be1.reshape(1, D)

    grid = (BN // BLK,)
    rep = lambda i: (0, 0)
    out = pl.pallas_call(
        _body,
        grid=grid,
        in_specs=[
            pl.BlockSpec((BLK, DMAX), lambda i: (i, 0)),
            pl.BlockSpec((BLK, 1), lambda i: (i, 0)),
            pl.BlockSpec((16, D), rep),
            pl.BlockSpec((DMAX, D), rep),
            pl.BlockSpec((D, D), rep),
            pl.BlockSpec((1, D), rep),
            pl.BlockSpec((1, D), rep),
            pl.BlockSpec((1, D), rep),
            pl.BlockSpec((1, D), rep),
            pl.BlockSpec((1, D), rep),
            pl.BlockSpec((1, D), rep),
        ],
        out_specs=pl.BlockSpec((BLK, D), lambda i: (i, 0)),
        out_shape=jax.ShapeDtypeStruct((BN, D), jnp.float32),
        compiler_params=pltpu.CompilerParams(
            dimension_semantics=("arbitrary",),
        ),
    )(x, t, wpb, W1, W2, cp, cb, c1, c2, g1r, be1r)
    return out.reshape(B, N, D)


# speedup vs baseline: 1.5214x; 1.2309x over previous
import jax
import jax.numpy as jnp
from jax.experimental import pallas as pl
from jax.experimental.pallas import tpu as pltpu

B, N, DMAX = 64, 2048, 64
D = 128


def _body(x_ref, t_ref, wpb_ref, w1_ref, w2_ref, cp_ref, cb_ref, c1_ref,
          c2_ref, g1_ref, be1_ref, o_ref):
    x = x_ref[0]
    t = t_ref[0, 0].reshape(N, 1)

    h = jnp.dot(x, w1_ref[:, :], preferred_element_type=jnp.float32)
    h = h + c1_ref[0, :]
    mu = jnp.mean(h, axis=-1, keepdims=True)
    var = jnp.mean((h - mu) ** 2, axis=-1, keepdims=True)
    h = (h - mu) * jax.lax.rsqrt(var + 1e-5) * g1_ref[0, :] + be1_ref[0, :]
    h = jnp.maximum(h, 0.0)
    poly = jnp.dot(h, w2_ref[:, :], preferred_element_type=jnp.float32)
    poly = poly + c2_ref[0, :]

    x8 = x[:, :8]
    wpb = wpb_ref[:, :]
    pt = jnp.dot(x8, wpb[:8, :], preferred_element_type=jnp.float32) + cp_ref[0, :]
    bx = jnp.dot(x8, wpb[8:, :], preferred_element_type=jnp.float32) + cb_ref[0, :]

    o_ref[0] = jnp.where(t == 0, pt, jnp.where(t == 1, bx, poly))


def kernel(prompts, prompt_types, Wp, bp, Wb, bb, W1, b1, g1, be1, W2, b2,
           type_emb):
    t3 = prompt_types.reshape(B, 1, N)

    wp8 = jnp.zeros((8, D), jnp.float32).at[:2, :].set(Wp)
    wb8 = jnp.zeros((8, D), jnp.float32).at[:4, :].set(Wb)
    wpb = jnp.concatenate([wp8, wb8], axis=0)

    cp = (bp + type_emb[0]).reshape(1, D)
    cb = (bb + type_emb[1]).reshape(1, D)
    c1 = b1.reshape(1, D)
    c2 = (b2 + type_emb[2]).reshape(1, D)
    g1r = g1.reshape(1, D)
    be1r = be1.reshape(1, D)

    rep = lambda i: (0, 0)
    out = pl.pallas_call(
        _body,
        grid=(B,),
        in_specs=[
            pl.BlockSpec((1, N, DMAX), lambda i: (i, 0, 0)),
            pl.BlockSpec((1, 1, N), lambda i: (i, 0, 0)),
            pl.BlockSpec((16, D), rep),
            pl.BlockSpec((DMAX, D), rep),
            pl.BlockSpec((D, D), rep),
            pl.BlockSpec((1, D), rep),
            pl.BlockSpec((1, D), rep),
            pl.BlockSpec((1, D), rep),
            pl.BlockSpec((1, D), rep),
            pl.BlockSpec((1, D), rep),
            pl.BlockSpec((1, D), rep),
        ],
        out_specs=pl.BlockSpec((1, N, D), lambda i: (i, 0, 0)),
        out_shape=jax.ShapeDtypeStruct((B, N, D), jnp.float32),
        compiler_params=pltpu.CompilerParams(
            dimension_semantics=("arbitrary",),
        ),
    )(prompts, t3, wpb, W1, W2, cp, cb, c1, c2, g1r, be1r)
    return out
